# baseline (speedup 1.0000x reference)
import functools

import jax
import jax.numpy as jnp
from jax import lax
from jax.experimental import pallas as pl
from jax.experimental.pallas import tpu as pltpu
from jax.experimental.pallas import tpu_sc as plsc

_B = 16384
_N_BASINS = 64
_NUM_CORES = 2
_NUM_SUBCORES = 16
_NW = _NUM_CORES * _NUM_SUBCORES
_CHUNK = _B // _NW
_L = 16


def _body(loss_hbm, idx_hbm, sigma_hbm, super_hbm, s_hbm,
          loss_v, idx_v, sigma_v, super_v, s_v, in_sem, out_sem):
    wid = lax.axis_index("s") * _NUM_CORES + lax.axis_index("c")
    base = wid * _CHUNK
    c_sigma = pltpu.async_copy(sigma_hbm, sigma_v, in_sem)
    c_loss = pltpu.async_copy(loss_hbm.at[pl.ds(base, _CHUNK)], loss_v, in_sem)
    c_idx = pltpu.async_copy(idx_hbm.at[pl.ds(base, _CHUNK)], idx_v, in_sem)
    c_sigma.wait()
    c_loss.wait()
    c_idx.wait()
    def step(i, _):
        sl = pl.ds(i * _L, _L)
        s = plsc.load_gather(sigma_v, [idx_v[sl]])
        s_v[sl] = s
        super_v[sl] = s * loss_v[sl]
        return _

    lax.fori_loop(0, _CHUNK // _L, step, None, unroll=4)
    c_super = pltpu.async_copy(super_v, super_hbm.at[pl.ds(base, _CHUNK)], out_sem)
    c_s = pltpu.async_copy(s_v, s_hbm.at[pl.ds(base, _CHUNK)], out_sem)
    c_super.wait()
    c_s.wait()


@jax.jit
def kernel(loss, basin_idx, sigma):
    mesh = plsc.VectorSubcoreMesh(core_axis_name="c", subcore_axis_name="s")
    f = functools.partial(
        pl.kernel,
        mesh=mesh,
        compiler_params=pltpu.CompilerParams(needs_layout_passes=False),
        out_type=[
            jax.ShapeDtypeStruct((_B,), jnp.float32),
            jax.ShapeDtypeStruct((_B,), jnp.float32),
        ],
        scratch_types=[
            pltpu.VMEM((_CHUNK,), jnp.float32),
            pltpu.VMEM((_CHUNK,), jnp.int32),
            pltpu.VMEM((_N_BASINS,), jnp.float32),
            pltpu.VMEM((_CHUNK,), jnp.float32),
            pltpu.VMEM((_CHUNK,), jnp.float32),
            pltpu.SemaphoreType.DMA,
            pltpu.SemaphoreType.DMA,
        ],
    )(_body)
    superloss, s = f(loss, basin_idx, sigma)
    return (superloss, s)


# speedup vs baseline: 1.0804x; 1.0804x over previous
import functools

import jax
import jax.numpy as jnp
from jax import lax
from jax.experimental import pallas as pl
from jax.experimental.pallas import tpu as pltpu
from jax.experimental.pallas import tpu_sc as plsc

_B = 16384
_N_BASINS = 64
_NUM_CORES = 1
_NUM_SUBCORES = 16
_NW = _NUM_CORES * _NUM_SUBCORES
_CHUNK = _B // _NW
_L = 16


def _body(loss_hbm, idx_hbm, sigma_hbm, super_hbm, s_hbm,
          loss_v, idx_v, sigma_v, super_v, s_v, in_sem, out_sem):
    wid = lax.axis_index("s") * _NUM_CORES + lax.axis_index("c")
    base = wid * _CHUNK
    c_sigma = pltpu.async_copy(sigma_hbm, sigma_v, in_sem)
    c_loss = pltpu.async_copy(loss_hbm.at[pl.ds(base, _CHUNK)], loss_v, in_sem)
    c_idx = pltpu.async_copy(idx_hbm.at[pl.ds(base, _CHUNK)], idx_v, in_sem)
    c_sigma.wait()
    c_loss.wait()
    c_idx.wait()
    def step(i, _):
        sl = pl.ds(i * _L, _L)
        s = plsc.load_gather(sigma_v, [idx_v[sl]])
        s_v[sl] = s
        super_v[sl] = s * loss_v[sl]
        return _

    lax.fori_loop(0, _CHUNK // _L, step, None, unroll=4)
    c_super = pltpu.async_copy(super_v, super_hbm.at[pl.ds(base, _CHUNK)], out_sem)
    c_s = pltpu.async_copy(s_v, s_hbm.at[pl.ds(base, _CHUNK)], out_sem)
    c_super.wait()
    c_s.wait()


@jax.jit
def kernel(loss, basin_idx, sigma):
    mesh = plsc.VectorSubcoreMesh(core_axis_name="c", subcore_axis_name="s", num_cores=_NUM_CORES)
    f = functools.partial(
        pl.kernel,
        mesh=mesh,
        compiler_params=pltpu.CompilerParams(needs_layout_passes=False),
        out_type=[
            jax.ShapeDtypeStruct((_B,), jnp.float32),
            jax.ShapeDtypeStruct((_B,), jnp.float32),
        ],
        scratch_types=[
            pltpu.VMEM((_CHUNK,), jnp.float32),
            pltpu.VMEM((_CHUNK,), jnp.int32),
            pltpu.VMEM((_N_BASINS,), jnp.float32),
            pltpu.VMEM((_CHUNK,), jnp.float32),
            pltpu.VMEM((_CHUNK,), jnp.float32),
            pltpu.SemaphoreType.DMA,
            pltpu.SemaphoreType.DMA,
        ],
    )(_body)
    superloss, s = f(loss, basin_idx, sigma)
    return (superloss, s)


# speedup vs baseline: 1.0812x; 1.0007x over previous
import functools

import jax
import jax.numpy as jnp
from jax import lax
from jax.experimental import pallas as pl
from jax.experimental.pallas import tpu as pltpu
from jax.experimental.pallas import tpu_sc as plsc

_B = 16384
_N_BASINS = 64
_NUM_CORES = 1
_NUM_SUBCORES = 16
_NW = _NUM_CORES * _NUM_SUBCORES
_CHUNK = _B // _NW
_L = 16


def _body(loss_hbm, idx_hbm, sigma_hbm, super_hbm, s_hbm,
          loss_v, idx_v, sigma_v, super_v, s_v, in_sem, out_sem):
    wid = lax.axis_index("s") * _NUM_CORES + lax.axis_index("c")
    base = wid * _CHUNK
    c_sigma = pltpu.async_copy(sigma_hbm, sigma_v, in_sem)
    c_loss = pltpu.async_copy(loss_hbm.at[pl.ds(base, _CHUNK)], loss_v, in_sem)
    c_idx = pltpu.async_copy(idx_hbm.at[pl.ds(base, _CHUNK)], idx_v, in_sem)
    c_sigma.wait()
    c_loss.wait()
    c_idx.wait()
    def step(i, _):
        sl = pl.ds(i * _L, _L)
        s = plsc.load_gather(sigma_v, [idx_v[sl]])
        s_v[sl] = s
        super_v[sl] = s * loss_v[sl]
        return _

    half = _CHUNK // 2
    n_half = half // _L
    lax.fori_loop(0, n_half, step, None, unroll=4)
    c_super0 = pltpu.async_copy(
        super_v.at[pl.ds(0, half)], super_hbm.at[pl.ds(base, half)], out_sem)
    c_s0 = pltpu.async_copy(
        s_v.at[pl.ds(0, half)], s_hbm.at[pl.ds(base, half)], out_sem)
    lax.fori_loop(n_half, 2 * n_half, step, None, unroll=4)
    c_super1 = pltpu.async_copy(
        super_v.at[pl.ds(half, half)], super_hbm.at[pl.ds(base + half, half)],
        out_sem)
    c_s1 = pltpu.async_copy(
        s_v.at[pl.ds(half, half)], s_hbm.at[pl.ds(base + half, half)], out_sem)
    c_super0.wait()
    c_s0.wait()
    c_super1.wait()
    c_s1.wait()


@jax.jit
def kernel(loss, basin_idx, sigma):
    mesh = plsc.VectorSubcoreMesh(core_axis_name="c", subcore_axis_name="s", num_cores=_NUM_CORES)
    f = functools.partial(
        pl.kernel,
        mesh=mesh,
        compiler_params=pltpu.CompilerParams(needs_layout_passes=False),
        out_type=[
            jax.ShapeDtypeStruct((_B,), jnp.float32),
            jax.ShapeDtypeStruct((_B,), jnp.float32),
        ],
        scratch_types=[
            pltpu.VMEM((_CHUNK,), jnp.float32),
            pltpu.VMEM((_CHUNK,), jnp.int32),
            pltpu.VMEM((_N_BASINS,), jnp.float32),
            pltpu.VMEM((_CHUNK,), jnp.float32),
            pltpu.VMEM((_CHUNK,), jnp.float32),
            pltpu.SemaphoreType.DMA,
            pltpu.SemaphoreType.DMA,
        ],
    )(_body)
    superloss, s = f(loss, basin_idx, sigma)
    return (superloss, s)
